# baseline (speedup 1.0000x reference)
import functools

import jax
import jax.numpy as jnp
from jax import lax
from jax.experimental import pallas as pl
from jax.experimental.pallas import tpu as pltpu
from jax.experimental.pallas import tpu_sc as plsc

N_NODES_C = 10000
D_C = 128
G_C = 64

NC = 2
NS = 16
K_EDGE = 128
N_PAD = 10240
ROWS_PER_TILE = N_PAD // NS

_MESH = plsc.VectorSubcoreMesh(core_axis_name="c", subcore_axis_name="s")


def _wid(c, s):
    return s * NC + c


def _sc_deg(dst3, ones_hbm, zeros16_hbm, n_chunks):
    @functools.partial(
        pl.kernel,
        out_type=jax.ShapeDtypeStruct((NC, N_PAD, D_C), jnp.float32),
        mesh=_MESH,
        scratch_types=[
            pltpu.VMEM((n_chunks, K_EDGE), jnp.int32),
            pltpu.VMEM((K_EDGE, D_C), jnp.float32),
            pltpu.VMEM_SHARED((N_PAD, D_C), jnp.float32),
        ],
    )
    def k(dst_hbm, ones_h, zeros_h, out_hbm, idx_v, ones_v, acc):
        c = lax.axis_index("c")
        s = lax.axis_index("s")
        w = _wid(c, s)
        row0 = s * ROWS_PER_TILE
        pltpu.sync_copy(zeros_h, acc.at[pl.ds(row0, ROWS_PER_TILE)])
        pltpu.sync_copy(ones_h, ones_v)
        pltpu.sync_copy(dst_hbm.at[w, pl.ds(0, n_chunks)], idx_v)
        plsc.subcore_barrier()

        def body(i):
            pltpu.sync_copy(ones_v, acc.at[idx_v.at[i]], add=True)

        pl.loop(0, n_chunks)(body)
        plsc.subcore_barrier()
        pltpu.sync_copy(acc.at[pl.ds(row0, ROWS_PER_TILE)],
                        out_hbm.at[c, pl.ds(row0, ROWS_PER_TILE)])

    return k(dst3, ones_hbm, zeros16_hbm)


EPC = 128


def _sc_agg(h, idx_flat, zerosd_hbm, nc0, nc1):
    blk = 2 * EPC

    @functools.partial(
        pl.kernel,
        out_type=jax.ShapeDtypeStruct((NC, N_PAD, D_C), jnp.float32),
        mesh=_MESH,
        scratch_types=[
            pltpu.VMEM((nc0 * blk,), jnp.int32),
            pltpu.VMEM((EPC, D_C), jnp.float32),
            pltpu.VMEM_SHARED((N_PAD, D_C), jnp.float32),
            pltpu.SemaphoreType.DMA,
        ],
    )
    def k(h_hbm, idx_hbm, zeros_h, out_hbm, idxv, stage, acc, sem):
        c = lax.axis_index("c")
        s = lax.axis_index("s")
        row0 = s * ROWS_PER_TILE
        base = jnp.where(c == 0, s * nc0 * blk,
                         (NS * nc0 + s * nc1) * blk)
        nc = jnp.where(c == 0, nc0, nc1)

        pltpu.sync_copy(zeros_h, acc.at[pl.ds(row0, ROWS_PER_TILE)])
        pltpu.sync_copy(idx_hbm.at[pl.ds(base, nc0 * blk)], idxv)
        plsc.subcore_barrier()

        def body(j):
            pltpu.async_copy(
                h_hbm.at[idxv.at[pl.ds(j * blk, EPC)]], stage, sem).wait()
            pltpu.sync_copy(
                stage, acc.at[idxv.at[pl.ds(j * blk + EPC, EPC)]], add=True)

        pl.loop(0, nc)(body)
        plsc.subcore_barrier()
        pltpu.sync_copy(acc.at[pl.ds(row0, ROWS_PER_TILE)],
                        out_hbm.at[c, pl.ds(row0, ROWS_PER_TILE)])

    return k(h, idx_flat, zerosd_hbm)


_BLK = 1024


def _dinv_from_parts(p_blk):
    deg = p_blk[0, :, 0:1] + p_blk[1, :, 0:1] + 1.0
    return lax.rsqrt(deg)


def _tc_first(deg_parts, x, W1):
    def body(dp_ref, x_ref, w_ref, o_ref):
        dinv = _dinv_from_parts(dp_ref[...])
        h = jnp.dot(x_ref[...], w_ref[...], preferred_element_type=jnp.float32)
        o_ref[...] = h * dinv

    return pl.pallas_call(
        body,
        out_shape=jax.ShapeDtypeStruct((N_PAD, D_C), jnp.float32),
        grid=(N_PAD // _BLK,),
        in_specs=[
            pl.BlockSpec((NC, _BLK, D_C), lambda i: (0, i, 0)),
            pl.BlockSpec((_BLK, D_C), lambda i: (i, 0)),
            pl.BlockSpec((D_C, D_C), lambda i: (0, 0)),
        ],
        out_specs=pl.BlockSpec((_BLK, D_C), lambda i: (i, 0)),
    )(deg_parts, x, W1)


def _tc_mid(parts, h_prev, deg_parts, W, b_prev):
    def body(p_ref, hp_ref, dp_ref, w_ref, b_ref, o_ref):
        dinv = _dinv_from_parts(dp_ref[...])
        agg = p_ref[0] + p_ref[1] + hp_ref[...]
        r = jnp.maximum(agg * dinv + b_ref[...], 0.0)
        o_ref[...] = jnp.dot(r, w_ref[...], preferred_element_type=jnp.float32) * dinv

    return pl.pallas_call(
        body,
        out_shape=jax.ShapeDtypeStruct((N_PAD, D_C), jnp.float32),
        grid=(N_PAD // _BLK,),
        in_specs=[
            pl.BlockSpec((NC, _BLK, D_C), lambda i: (0, i, 0)),
            pl.BlockSpec((_BLK, D_C), lambda i: (i, 0)),
            pl.BlockSpec((NC, _BLK, D_C), lambda i: (0, i, 0)),
            pl.BlockSpec((D_C, D_C), lambda i: (0, 0)),
            pl.BlockSpec((1, D_C), lambda i: (0, 0)),
        ],
        out_specs=pl.BlockSpec((_BLK, D_C), lambda i: (i, 0)),
    )(parts, h_prev, deg_parts, W, b_prev.reshape(1, D_C))


def _tc_final(parts, h_prev, deg_parts, b3, batch2d):
    ngrid = N_PAD // _BLK

    def body(p_ref, hp_ref, dp_ref, b_ref, bat_ref, o_ref, sums, counts):
        i = pl.program_id(0)

        @pl.when(i == 0)
        def _():
            sums[...] = jnp.zeros_like(sums)
            counts[...] = jnp.zeros_like(counts)

        dinv = _dinv_from_parts(dp_ref[...])
        out3 = (p_ref[0] + p_ref[1] + hp_ref[...]) * dinv + b_ref[...]
        lane = lax.broadcasted_iota(jnp.int32, (_BLK, D_C), 1)
        onehot = jnp.where((bat_ref[...] == lane) & (lane < G_C), 1.0, 0.0)
        sums[...] += lax.dot_general(
            onehot, out3, (((0,), (0,)), ((), ())),
            preferred_element_type=jnp.float32)
        counts[...] += lax.dot_general(
            onehot, jnp.ones((_BLK, D_C), jnp.float32), (((0,), (0,)), ((), ())),
            preferred_element_type=jnp.float32)

        @pl.when(i == ngrid - 1)
        def _():
            o_ref[...] = sums[0:G_C] / jnp.maximum(counts[0:G_C], 1.0)

    return pl.pallas_call(
        body,
        out_shape=jax.ShapeDtypeStruct((G_C, D_C), jnp.float32),
        grid=(ngrid,),
        in_specs=[
            pl.BlockSpec((NC, _BLK, D_C), lambda i: (0, i, 0)),
            pl.BlockSpec((_BLK, D_C), lambda i: (i, 0)),
            pl.BlockSpec((NC, _BLK, D_C), lambda i: (0, i, 0)),
            pl.BlockSpec((1, D_C), lambda i: (0, 0)),
            pl.BlockSpec((_BLK, D_C), lambda i: (i, 0)),
        ],
        out_specs=pl.BlockSpec((G_C, D_C), lambda i: (0, 0)),
        scratch_shapes=[
            pltpu.VMEM((D_C, D_C), jnp.float32),
            pltpu.VMEM((D_C, D_C), jnp.float32),
        ],
        compiler_params=pltpu.CompilerParams(
            dimension_semantics=("arbitrary",)),
    )(parts, h_prev, deg_parts, b3.reshape(1, D_C), batch2d)


def kernel(x, edge_index, batch_index, W1, b1, W2, b2, W3, b3):
    n, d = x.shape
    e = edge_index.shape[1]
    tiles = NC * NS

    n_deg = -(-e // (tiles * K_EDGE))
    n_deg = -(-n_deg // 8) * 8
    e_pad = tiles * n_deg * K_EDGE
    dst = jnp.concatenate(
        [edge_index[1], jnp.full((e_pad - e,), n, jnp.int32)]).reshape(
            tiles, n_deg, K_EDGE)

    frac0 = 0.70
    nc0 = -(-int(e * frac0) // (NS * EPC))
    nc1 = -(-(e - NS * EPC * nc0) // (NS * EPC))
    e0 = NS * EPC * nc0
    e_pad2 = e0 + NS * EPC * nc1
    src2 = jnp.concatenate(
        [edge_index[0], jnp.zeros((e_pad2 - e,), jnp.int32)])
    dst2 = jnp.concatenate(
        [edge_index[1], jnp.full((e_pad2 - e,), n, jnp.int32)])

    def interleave(sv, dv, nc):
        m = sv.shape[0] // (nc * EPC)
        return jnp.stack([sv.reshape(m, nc, EPC), dv.reshape(m, nc, EPC)],
                         axis=2).reshape(-1)

    idx_flat = jnp.concatenate([
        interleave(src2[:e0], dst2[:e0], nc0),
        interleave(src2[e0:], dst2[e0:], nc1),
        jnp.zeros(((nc0 - nc1) * 2 * EPC,), jnp.int32),
    ])

    x_pad = jnp.zeros((N_PAD, d), x.dtype).at[0:n].set(x)
    batch_pad = jnp.full((N_PAD,), G_C, jnp.int32).at[0:n].set(batch_index)
    batch2d = jnp.broadcast_to(batch_pad[:, None], (N_PAD, D_C))

    ones128 = jnp.ones((K_EDGE, D_C), jnp.float32)

    zerosd = jnp.zeros((ROWS_PER_TILE, D_C), jnp.float32)

    deg_parts = _sc_deg(dst, ones128, zerosd, n_deg)
    h1 = _tc_first(deg_parts, x_pad, W1)
    p1 = _sc_agg(h1, idx_flat, zerosd, nc0, nc1)
    h2 = _tc_mid(p1, h1, deg_parts, W2, b1)
    p2 = _sc_agg(h2, idx_flat, zerosd, nc0, nc1)
    h3 = _tc_mid(p2, h2, deg_parts, W3, b2)
    p3 = _sc_agg(h3, idx_flat, zerosd, nc0, nc1)
    return _tc_final(p3, h3, deg_parts, b3, batch2d)


# speedup vs baseline: 1.0906x; 1.0906x over previous
import functools

import jax
import jax.numpy as jnp
from jax import lax
from jax.experimental import pallas as pl
from jax.experimental.pallas import tpu as pltpu
from jax.experimental.pallas import tpu_sc as plsc

N_NODES_C = 10000
D_C = 128
G_C = 64

NC = 2
NS = 16
K_EDGE = 128
N_PAD = 10240
ROWS_PER_TILE = N_PAD // NS

_MESH = plsc.VectorSubcoreMesh(core_axis_name="c", subcore_axis_name="s")


def _wid(c, s):
    return s * NC + c


def _sc_deg(dst3, ones_hbm, zeros16_hbm, n_chunks):
    @functools.partial(
        pl.kernel,
        out_type=jax.ShapeDtypeStruct((NC, N_PAD, D_C), jnp.float32),
        mesh=_MESH,
        scratch_types=[
            pltpu.VMEM((n_chunks, K_EDGE), jnp.int32),
            pltpu.VMEM((K_EDGE, D_C), jnp.float32),
            pltpu.VMEM_SHARED((N_PAD, D_C), jnp.float32),
        ],
    )
    def k(dst_hbm, ones_h, zeros_h, out_hbm, idx_v, ones_v, acc):
        c = lax.axis_index("c")
        s = lax.axis_index("s")
        w = _wid(c, s)
        row0 = s * ROWS_PER_TILE
        pltpu.sync_copy(zeros_h, acc.at[pl.ds(row0, ROWS_PER_TILE)])
        pltpu.sync_copy(ones_h, ones_v)
        pltpu.sync_copy(dst_hbm.at[w, pl.ds(0, n_chunks)], idx_v)
        plsc.subcore_barrier()

        def body(i):
            pltpu.sync_copy(ones_v, acc.at[idx_v.at[i]], add=True)

        pl.loop(0, n_chunks)(body)
        plsc.subcore_barrier()
        pltpu.sync_copy(acc.at[pl.ds(row0, ROWS_PER_TILE)],
                        out_hbm.at[c, pl.ds(row0, ROWS_PER_TILE)])

    return k(dst3, ones_hbm, zeros16_hbm)


EPC = 128


def _sc_agg(h, idx_flat, zerosd_hbm, nc0, nc1):
    blk = 2 * EPC

    @functools.partial(
        pl.kernel,
        out_type=jax.ShapeDtypeStruct((NC, N_PAD, D_C), jnp.float32),
        mesh=_MESH,
        scratch_types=[
            pltpu.VMEM((nc0 * blk,), jnp.int32),
            pltpu.VMEM((EPC, D_C), jnp.float32),
            pltpu.VMEM_SHARED((N_PAD, D_C), jnp.float32),
            pltpu.SemaphoreType.DMA,
        ],
    )
    def k(h_hbm, idx_hbm, zeros_h, out_hbm, idxv, stage, acc, sem):
        c = lax.axis_index("c")
        s = lax.axis_index("s")
        row0 = s * ROWS_PER_TILE
        base = jnp.where(c == 0, s * nc0 * blk,
                         (NS * nc0 + s * nc1) * blk)
        nc = jnp.where(c == 0, nc0, nc1)

        pltpu.sync_copy(zeros_h, acc.at[pl.ds(row0, ROWS_PER_TILE)])
        pltpu.sync_copy(idx_hbm.at[pl.ds(base, nc0 * blk)], idxv)
        plsc.subcore_barrier()

        def body(j):
            pltpu.async_copy(
                h_hbm.at[idxv.at[pl.ds(j * blk, EPC)]], stage, sem).wait()
            pltpu.sync_copy(
                stage, acc.at[idxv.at[pl.ds(j * blk + EPC, EPC)]], add=True)

        pl.loop(0, nc)(body)
        plsc.subcore_barrier()
        pltpu.sync_copy(acc.at[pl.ds(row0, ROWS_PER_TILE)],
                        out_hbm.at[c, pl.ds(row0, ROWS_PER_TILE)])

    return k(h, idx_flat, zerosd_hbm)


_BLK = 1024


def _dinv_from_parts(p_blk):
    deg = p_blk[0, :, 0:1] + p_blk[1, :, 0:1] + 1.0
    return lax.rsqrt(deg)


def _tc_first(deg_parts, x, W1):
    def body(dp_ref, x_ref, w_ref, o_ref):
        dinv = _dinv_from_parts(dp_ref[...])
        h = jnp.dot(x_ref[...], w_ref[...], preferred_element_type=jnp.float32)
        o_ref[...] = h * dinv

    return pl.pallas_call(
        body,
        out_shape=jax.ShapeDtypeStruct((N_PAD, D_C), jnp.float32),
        grid=(N_PAD // _BLK,),
        in_specs=[
            pl.BlockSpec((NC, _BLK, D_C), lambda i: (0, i, 0)),
            pl.BlockSpec((_BLK, D_C), lambda i: (i, 0)),
            pl.BlockSpec((D_C, D_C), lambda i: (0, 0)),
        ],
        out_specs=pl.BlockSpec((_BLK, D_C), lambda i: (i, 0)),
    )(deg_parts, x, W1)


def _tc_mid(parts, h_prev, deg_parts, W, b_prev):
    def body(p_ref, hp_ref, dp_ref, w_ref, b_ref, o_ref):
        dinv = _dinv_from_parts(dp_ref[...])
        agg = p_ref[0] + p_ref[1] + hp_ref[...]
        r = jnp.maximum(agg * dinv + b_ref[...], 0.0)
        o_ref[...] = jnp.dot(r, w_ref[...], preferred_element_type=jnp.float32) * dinv

    return pl.pallas_call(
        body,
        out_shape=jax.ShapeDtypeStruct((N_PAD, D_C), jnp.float32),
        grid=(N_PAD // _BLK,),
        in_specs=[
            pl.BlockSpec((NC, _BLK, D_C), lambda i: (0, i, 0)),
            pl.BlockSpec((_BLK, D_C), lambda i: (i, 0)),
            pl.BlockSpec((NC, _BLK, D_C), lambda i: (0, i, 0)),
            pl.BlockSpec((D_C, D_C), lambda i: (0, 0)),
            pl.BlockSpec((1, D_C), lambda i: (0, 0)),
        ],
        out_specs=pl.BlockSpec((_BLK, D_C), lambda i: (i, 0)),
    )(parts, h_prev, deg_parts, W, b_prev.reshape(1, D_C))


def _tc_final(parts, h_prev, deg_parts, b3, batch2d):
    ngrid = N_PAD // _BLK

    def body(p_ref, hp_ref, dp_ref, b_ref, bat_ref, o_ref, sums, counts):
        i = pl.program_id(0)

        @pl.when(i == 0)
        def _():
            sums[...] = jnp.zeros_like(sums)
            counts[...] = jnp.zeros_like(counts)

        dinv = _dinv_from_parts(dp_ref[...])
        out3 = (p_ref[0] + p_ref[1] + hp_ref[...]) * dinv + b_ref[...]
        lane = lax.broadcasted_iota(jnp.int32, (_BLK, D_C), 1)
        onehot = jnp.where((bat_ref[...] == lane) & (lane < G_C), 1.0, 0.0)
        sums[...] += lax.dot_general(
            onehot, out3, (((0,), (0,)), ((), ())),
            preferred_element_type=jnp.float32)
        counts[...] += lax.dot_general(
            onehot, jnp.ones((_BLK, D_C), jnp.float32), (((0,), (0,)), ((), ())),
            preferred_element_type=jnp.float32)

        @pl.when(i == ngrid - 1)
        def _():
            o_ref[...] = sums[0:G_C] / jnp.maximum(counts[0:G_C], 1.0)

    return pl.pallas_call(
        body,
        out_shape=jax.ShapeDtypeStruct((G_C, D_C), jnp.float32),
        grid=(ngrid,),
        in_specs=[
            pl.BlockSpec((NC, _BLK, D_C), lambda i: (0, i, 0)),
            pl.BlockSpec((_BLK, D_C), lambda i: (i, 0)),
            pl.BlockSpec((NC, _BLK, D_C), lambda i: (0, i, 0)),
            pl.BlockSpec((1, D_C), lambda i: (0, 0)),
            pl.BlockSpec((_BLK, D_C), lambda i: (i, 0)),
        ],
        out_specs=pl.BlockSpec((G_C, D_C), lambda i: (0, 0)),
        scratch_shapes=[
            pltpu.VMEM((D_C, D_C), jnp.float32),
            pltpu.VMEM((D_C, D_C), jnp.float32),
        ],
        compiler_params=pltpu.CompilerParams(
            dimension_semantics=("arbitrary",)),
    )(parts, h_prev, deg_parts, b3.reshape(1, D_C), batch2d)


def kernel(x, edge_index, batch_index, W1, b1, W2, b2, W3, b3):
    n, d = x.shape
    e = edge_index.shape[1]
    tiles = NC * NS

    n_deg = -(-e // (tiles * K_EDGE))
    n_deg = -(-n_deg // 8) * 8
    e_pad = tiles * n_deg * K_EDGE
    dst = jnp.concatenate(
        [edge_index[1], jnp.full((e_pad - e,), n, jnp.int32)]).reshape(
            tiles, n_deg, K_EDGE)

    frac0 = 0.62
    nc0 = -(-int(e * frac0) // (NS * EPC))
    nc1 = -(-(e - NS * EPC * nc0) // (NS * EPC))
    e0 = NS * EPC * nc0
    e_pad2 = e0 + NS * EPC * nc1
    src2 = jnp.concatenate(
        [edge_index[0], jnp.zeros((e_pad2 - e,), jnp.int32)])
    dst2 = jnp.concatenate(
        [edge_index[1], jnp.full((e_pad2 - e,), n, jnp.int32)])

    def interleave(sv, dv, nc):
        m = sv.shape[0] // (nc * EPC)
        return jnp.stack([sv.reshape(m, nc, EPC), dv.reshape(m, nc, EPC)],
                         axis=2).reshape(-1)

    idx_flat = jnp.concatenate([
        interleave(src2[:e0], dst2[:e0], nc0),
        interleave(src2[e0:], dst2[e0:], nc1),
        jnp.zeros(((nc0 - nc1) * 2 * EPC,), jnp.int32),
    ])

    x_pad = jnp.zeros((N_PAD, d), x.dtype).at[0:n].set(x)
    batch_pad = jnp.full((N_PAD,), G_C, jnp.int32).at[0:n].set(batch_index)
    batch2d = jnp.broadcast_to(batch_pad[:, None], (N_PAD, D_C))

    ones128 = jnp.ones((K_EDGE, D_C), jnp.float32)

    zerosd = jnp.zeros((ROWS_PER_TILE, D_C), jnp.float32)

    deg_parts = _sc_deg(dst, ones128, zerosd, n_deg)
    h1 = _tc_first(deg_parts, x_pad, W1)
    p1 = _sc_agg(h1, idx_flat, zerosd, nc0, nc1)
    h2 = _tc_mid(p1, h1, deg_parts, W2, b1)
    p2 = _sc_agg(h2, idx_flat, zerosd, nc0, nc1)
    h3 = _tc_mid(p2, h2, deg_parts, W3, b2)
    p3 = _sc_agg(h3, idx_flat, zerosd, nc0, nc1)
    return _tc_final(p3, h3, deg_parts, b3, batch2d)


# speedup vs baseline: 1.0967x; 1.0056x over previous
import functools

import jax
import jax.numpy as jnp
from jax import lax
from jax.experimental import pallas as pl
from jax.experimental.pallas import tpu as pltpu
from jax.experimental.pallas import tpu_sc as plsc

N_NODES_C = 10000
D_C = 128
G_C = 64

NC = 2
NS = 16
K_EDGE = 128
N_PAD = 10240
ROWS_PER_TILE = N_PAD // NS

_MESH = plsc.VectorSubcoreMesh(core_axis_name="c", subcore_axis_name="s")


def _wid(c, s):
    return s * NC + c


def _sc_deg(dst3, ones_hbm, zeros16_hbm, n_chunks):
    @functools.partial(
        pl.kernel,
        out_type=jax.ShapeDtypeStruct((NC, N_PAD, D_C), jnp.float32),
        mesh=_MESH,
        scratch_types=[
            pltpu.VMEM((n_chunks, K_EDGE), jnp.int32),
            pltpu.VMEM((K_EDGE, D_C), jnp.float32),
            pltpu.VMEM_SHARED((N_PAD, D_C), jnp.float32),
        ],
    )
    def k(dst_hbm, ones_h, zeros_h, out_hbm, idx_v, ones_v, acc):
        c = lax.axis_index("c")
        s = lax.axis_index("s")
        w = _wid(c, s)
        row0 = s * ROWS_PER_TILE
        pltpu.sync_copy(zeros_h, acc.at[pl.ds(row0, ROWS_PER_TILE)])
        pltpu.sync_copy(ones_h, ones_v)
        pltpu.sync_copy(dst_hbm.at[w, pl.ds(0, n_chunks)], idx_v)
        plsc.subcore_barrier()

        def body(i):
            pltpu.sync_copy(ones_v, acc.at[idx_v.at[i]], add=True)

        pl.loop(0, n_chunks)(body)
        plsc.subcore_barrier()
        pltpu.sync_copy(acc.at[pl.ds(row0, ROWS_PER_TILE)],
                        out_hbm.at[c, pl.ds(row0, ROWS_PER_TILE)])

    return k(dst3, ones_hbm, zeros16_hbm)


EPC = 128


def _sc_agg(h, idx_flat, zerosd_hbm, nc0, nc1):
    blk = 2 * EPC

    @functools.partial(
        pl.kernel,
        out_type=jax.ShapeDtypeStruct((NC, N_PAD, D_C), jnp.float32),
        mesh=_MESH,
        scratch_types=[
            pltpu.VMEM((nc0 * blk,), jnp.int32),
            pltpu.VMEM((EPC, D_C), jnp.float32),
            pltpu.VMEM_SHARED((N_PAD, D_C), jnp.float32),
            pltpu.SemaphoreType.DMA,
        ],
    )
    def k(h_hbm, idx_hbm, zeros_h, out_hbm, idxv, stage, acc, sem):
        c = lax.axis_index("c")
        s = lax.axis_index("s")
        row0 = s * ROWS_PER_TILE
        base = jnp.where(c == 0, s * nc0 * blk,
                         (NS * nc0 + s * nc1) * blk)
        nc = jnp.where(c == 0, nc0, nc1)

        pltpu.sync_copy(zeros_h, acc.at[pl.ds(row0, ROWS_PER_TILE)])
        pltpu.sync_copy(idx_hbm.at[pl.ds(base, nc0 * blk)], idxv)
        plsc.subcore_barrier()

        def body(j):
            pltpu.async_copy(
                h_hbm.at[idxv.at[pl.ds(j * blk, EPC)]], stage, sem).wait()
            pltpu.sync_copy(
                stage, acc.at[idxv.at[pl.ds(j * blk + EPC, EPC)]], add=True)

        pl.loop(0, nc)(body)
        plsc.subcore_barrier()
        pltpu.sync_copy(acc.at[pl.ds(row0, ROWS_PER_TILE)],
                        out_hbm.at[c, pl.ds(row0, ROWS_PER_TILE)])

    return k(h, idx_flat, zerosd_hbm)


_BLK = 1024


def _dinv_from_parts(p_blk):
    deg = p_blk[0, :, 0:1] + p_blk[1, :, 0:1] + 1.0
    return lax.rsqrt(deg)


def _tc_first(deg_parts, x, W1):
    def body(dp_ref, x_ref, w_ref, o_ref):
        dinv = _dinv_from_parts(dp_ref[...])
        h = jnp.dot(x_ref[...], w_ref[...], preferred_element_type=jnp.float32)
        o_ref[...] = h * dinv

    return pl.pallas_call(
        body,
        out_shape=jax.ShapeDtypeStruct((N_PAD, D_C), jnp.float32),
        grid=(N_PAD // _BLK,),
        in_specs=[
            pl.BlockSpec((NC, _BLK, D_C), lambda i: (0, i, 0)),
            pl.BlockSpec((_BLK, D_C), lambda i: (i, 0)),
            pl.BlockSpec((D_C, D_C), lambda i: (0, 0)),
        ],
        out_specs=pl.BlockSpec((_BLK, D_C), lambda i: (i, 0)),
    )(deg_parts, x, W1)


def _tc_mid(parts, h_prev, deg_parts, W, b_prev):
    def body(p_ref, hp_ref, dp_ref, w_ref, b_ref, o_ref):
        dinv = _dinv_from_parts(dp_ref[...])
        agg = p_ref[0] + p_ref[1] + hp_ref[...]
        r = jnp.maximum(agg * dinv + b_ref[...], 0.0)
        o_ref[...] = jnp.dot(r, w_ref[...], preferred_element_type=jnp.float32) * dinv

    return pl.pallas_call(
        body,
        out_shape=jax.ShapeDtypeStruct((N_PAD, D_C), jnp.float32),
        grid=(N_PAD // _BLK,),
        in_specs=[
            pl.BlockSpec((NC, _BLK, D_C), lambda i: (0, i, 0)),
            pl.BlockSpec((_BLK, D_C), lambda i: (i, 0)),
            pl.BlockSpec((NC, _BLK, D_C), lambda i: (0, i, 0)),
            pl.BlockSpec((D_C, D_C), lambda i: (0, 0)),
            pl.BlockSpec((1, D_C), lambda i: (0, 0)),
        ],
        out_specs=pl.BlockSpec((_BLK, D_C), lambda i: (i, 0)),
    )(parts, h_prev, deg_parts, W, b_prev.reshape(1, D_C))


def _tc_final(parts, h_prev, deg_parts, b3, batch2d):
    ngrid = N_PAD // _BLK

    def body(p_ref, hp_ref, dp_ref, b_ref, bat_ref, o_ref, sums, counts):
        i = pl.program_id(0)

        @pl.when(i == 0)
        def _():
            sums[...] = jnp.zeros_like(sums)
            counts[...] = jnp.zeros_like(counts)

        dinv = _dinv_from_parts(dp_ref[...])
        out3 = (p_ref[0] + p_ref[1] + hp_ref[...]) * dinv + b_ref[...]
        lane = lax.broadcasted_iota(jnp.int32, (_BLK, D_C), 1)
        onehot = jnp.where((bat_ref[...] == lane) & (lane < G_C), 1.0, 0.0)
        sums[...] += lax.dot_general(
            onehot, out3, (((0,), (0,)), ((), ())),
            preferred_element_type=jnp.float32)
        counts[...] += lax.dot_general(
            onehot, jnp.ones((_BLK, D_C), jnp.float32), (((0,), (0,)), ((), ())),
            preferred_element_type=jnp.float32)

        @pl.when(i == ngrid - 1)
        def _():
            o_ref[...] = sums[0:G_C] / jnp.maximum(counts[0:G_C], 1.0)

    return pl.pallas_call(
        body,
        out_shape=jax.ShapeDtypeStruct((G_C, D_C), jnp.float32),
        grid=(ngrid,),
        in_specs=[
            pl.BlockSpec((NC, _BLK, D_C), lambda i: (0, i, 0)),
            pl.BlockSpec((_BLK, D_C), lambda i: (i, 0)),
            pl.BlockSpec((NC, _BLK, D_C), lambda i: (0, i, 0)),
            pl.BlockSpec((1, D_C), lambda i: (0, 0)),
            pl.BlockSpec((_BLK, D_C), lambda i: (i, 0)),
        ],
        out_specs=pl.BlockSpec((G_C, D_C), lambda i: (0, 0)),
        scratch_shapes=[
            pltpu.VMEM((D_C, D_C), jnp.float32),
            pltpu.VMEM((D_C, D_C), jnp.float32),
        ],
        compiler_params=pltpu.CompilerParams(
            dimension_semantics=("arbitrary",)),
    )(parts, h_prev, deg_parts, b3.reshape(1, D_C), batch2d)


def kernel(x, edge_index, batch_index, W1, b1, W2, b2, W3, b3):
    n, d = x.shape
    e = edge_index.shape[1]
    tiles = NC * NS

    n_deg = -(-e // (tiles * K_EDGE))
    n_deg = -(-n_deg // 8) * 8
    e_pad = tiles * n_deg * K_EDGE
    dst = jnp.concatenate(
        [edge_index[1], jnp.full((e_pad - e,), n, jnp.int32)]).reshape(
            tiles, n_deg, K_EDGE)

    frac0 = 0.60
    nc0 = -(-int(e * frac0) // (NS * EPC))
    nc1 = -(-(e - NS * EPC * nc0) // (NS * EPC))
    e0 = NS * EPC * nc0
    e_pad2 = e0 + NS * EPC * nc1
    src2 = jnp.concatenate(
        [edge_index[0], jnp.zeros((e_pad2 - e,), jnp.int32)])
    dst2 = jnp.concatenate(
        [edge_index[1], jnp.full((e_pad2 - e,), n, jnp.int32)])

    def interleave(sv, dv, nc):
        m = sv.shape[0] // (nc * EPC)
        return jnp.stack([sv.reshape(m, nc, EPC), dv.reshape(m, nc, EPC)],
                         axis=2).reshape(-1)

    idx_flat = jnp.concatenate([
        interleave(src2[:e0], dst2[:e0], nc0),
        interleave(src2[e0:], dst2[e0:], nc1),
        jnp.zeros(((nc0 - nc1) * 2 * EPC,), jnp.int32),
    ])

    x_pad = jnp.zeros((N_PAD, d), x.dtype).at[0:n].set(x)
    batch_pad = jnp.full((N_PAD,), G_C, jnp.int32).at[0:n].set(batch_index)
    batch2d = jnp.broadcast_to(batch_pad[:, None], (N_PAD, D_C))

    ones128 = jnp.ones((K_EDGE, D_C), jnp.float32)

    zerosd = jnp.zeros((ROWS_PER_TILE, D_C), jnp.float32)

    deg_parts = _sc_deg(dst, ones128, zerosd, n_deg)
    h1 = _tc_first(deg_parts, x_pad, W1)
    p1 = _sc_agg(h1, idx_flat, zerosd, nc0, nc1)
    h2 = _tc_mid(p1, h1, deg_parts, W2, b1)
    p2 = _sc_agg(h2, idx_flat, zerosd, nc0, nc1)
    h3 = _tc_mid(p2, h2, deg_parts, W3, b2)
    p3 = _sc_agg(h3, idx_flat, zerosd, nc0, nc1)
    return _tc_final(p3, h3, deg_parts, b3, batch2d)
